# baseline (speedup 1.0000x reference)
import functools
import jax
import jax.numpy as jnp
from jax import lax
from jax.experimental import pallas as pl
from jax.experimental.pallas import tpu as pltpu, tpu_sc as plsc

_SEQ, _HID, _FF, _E = 2048, 1024, 2048, 8
_CHUNK = _SEQ // _E
_LANES = 128
_FT = 512
_FN = _FF // _FT


def _make_sc_router():
    info = plsc.get_sparse_core_info()
    nc, ns, lanes = info.num_cores, info.num_subcores, info.num_lanes
    nw = nc * ns
    tok_w = _SEQ // nw
    ngrp = tok_w // lanes
    mesh = plsc.VectorSubcoreMesh(core_axis_name="c", subcore_axis_name="s")

    @functools.partial(
        pl.kernel, mesh=mesh,
        out_type=jax.ShapeDtypeStruct((nw, _E * tok_w), jnp.float32),
        scratch_types=[
            pltpu.VMEM((_E * _SEQ,), jnp.float32),
            pltpu.VMEM((_E * tok_w,), jnp.float32),
        ],
    )
    def sc_router(lg_hbm, out_hbm, lg_v, sc_v):
        wid = lax.axis_index("s") * nc + lax.axis_index("c")
        base = wid * tok_w
        pltpu.sync_copy(lg_hbm, lg_v)
        for j in range(ngrp):
            le = []
            for e in range(_E):
                le.append(lg_v[pl.ds(e * _SEQ + base + j * lanes, lanes)])
            m1 = le[0]
            for e in range(1, _E):
                m1 = jnp.maximum(m1, le[e])
            big = lax.full((lanes,), 99, jnp.int32)
            i1 = big
            for e in range(_E):
                i1 = jnp.minimum(i1, jnp.where(le[e] == m1, e, big))
            neg = lax.full((lanes,), -jnp.inf, jnp.float32)
            l2 = [jnp.where(i1 == e, neg, le[e]) for e in range(_E)]
            m2 = l2[0]
            for e in range(1, _E):
                m2 = jnp.maximum(m2, l2[e])
            i2 = big
            for e in range(_E):
                i2 = jnp.minimum(i2, jnp.where(l2[e] == m2, e, big))
            for e in range(_E):
                sig = 1.0 / (1.0 + jnp.exp(-le[e]))
                sel = (i1 == e) | (i2 == e)
                sc_v[pl.ds(e * tok_w + j * lanes, lanes)] = \
                    jnp.where(sel, sig, 0.0)
        pltpu.sync_copy(sc_v, out_hbm.at[wid])

    return sc_router


_sc_router = _make_sc_router()


def _moe_body(x_ref, wr_ref, wg_ref, wu_ref, wsg_ref, wsu_ref, wd_ref,
              wsd_ref, out_ref):
    c = pl.program_id(0)
    f = pl.program_id(1)
    rows = pl.ds(c * _CHUNK, _CHUNK)
    cols = pl.ds(f * _FT, _FT)

    x16 = x_ref[rows, :].astype(jnp.bfloat16)

    logits = jnp.dot(x16, wr_ref[...], preferred_element_type=jnp.float32)
    lane = jax.lax.broadcasted_iota(jnp.int32, logits.shape, 1)
    neg_inf = jnp.float32(-jnp.inf)
    l = jnp.where(lane < _E, logits, neg_inf)
    v1 = jnp.max(l, axis=1, keepdims=True)
    i1 = jnp.min(jnp.where(l == v1, lane, _LANES), axis=1, keepdims=True)
    l2 = jnp.where(lane == i1, neg_inf, l)
    v2 = jnp.max(l2, axis=1, keepdims=True)
    s1 = jax.nn.sigmoid(v1)
    s2 = jax.nn.sigmoid(v2)

    g = jnp.dot(x16, wg_ref[0].astype(jnp.bfloat16),
                preferred_element_type=jnp.float32)
    u = jnp.dot(x16, wu_ref[0].astype(jnp.bfloat16),
                preferred_element_type=jnp.float32)
    gs = jnp.dot(x16, wsg_ref[:, cols].astype(jnp.bfloat16),
                 preferred_element_type=jnp.float32)
    us = jnp.dot(x16, wsu_ref[:, cols].astype(jnp.bfloat16),
                 preferred_element_type=jnp.float32)

    a1 = s1 * g
    a2 = s2 * g
    h_routed = (s1 * u) * (a1 * jax.nn.sigmoid(a1)) \
             + (s2 * u) * (a2 * jax.nn.sigmoid(a2))
    h_shared = us * (gs * jax.nn.sigmoid(gs))

    partial = (
        jnp.dot(h_routed.astype(jnp.bfloat16), wd_ref[0].astype(jnp.bfloat16),
                preferred_element_type=jnp.float32)
        + jnp.dot(h_shared.astype(jnp.bfloat16),
                  wsd_ref[cols, :].astype(jnp.bfloat16),
                  preferred_element_type=jnp.float32))

    @pl.when(f == 0)
    def _():
        out_ref[...] = partial

    @pl.when(f > 0)
    def _():
        out_ref[...] += partial


def kernel(hidden_states, router_kernel, gate_up_proj, down_proj,
           shared_gate_kernel, shared_up_kernel, shared_down_kernel):
    batch, seq, hid = hidden_states.shape
    flat = hidden_states.reshape(seq, hid)
    router_logits = flat @ router_kernel
    sc_raw = _sc_router(router_logits.T.reshape(-1))
    scores_t = sc_raw.reshape(32, _E, _SEQ // 32).transpose(1, 0, 2) \
                     .reshape(_E, _SEQ)

    wr16 = jnp.pad(router_kernel, ((0, 0), (0, _LANES - _E))
                   ).astype(jnp.bfloat16)

    out = pl.pallas_call(
        _moe_body,
        grid=(_E, _FN),
        in_specs=[
            pl.BlockSpec((_SEQ, _HID), lambda c, f: (0, 0)),
            pl.BlockSpec((_HID, _LANES), lambda c, f: (0, 0)),
            pl.BlockSpec((1, _HID, _FT), lambda c, f: (c, 0, f)),
            pl.BlockSpec((1, _HID, _FT), lambda c, f: (c, 0, f + _FN)),
            pl.BlockSpec((_HID, _FF), lambda c, f: (0, 0)),
            pl.BlockSpec((_HID, _FF), lambda c, f: (0, 0)),
            pl.BlockSpec((1, _FT, _HID), lambda c, f: (c, f, 0)),
            pl.BlockSpec((_FF, _HID), lambda c, f: (0, 0)),
        ],
        out_specs=pl.BlockSpec((_CHUNK, _HID), lambda c, f: (c, 0)),
        out_shape=jax.ShapeDtypeStruct((seq, hid), jnp.float32),
        compiler_params=pltpu.CompilerParams(
            dimension_semantics=("parallel", "arbitrary")),
    )(flat, wr16, gate_up_proj, gate_up_proj,
      shared_gate_kernel, shared_up_kernel, down_proj, shared_down_kernel)

    return out.reshape(batch, seq, hid), scores_t


# speedup vs baseline: 1.1969x; 1.1969x over previous
import jax
import jax.numpy as jnp
from jax.experimental import pallas as pl
from jax.experimental.pallas import tpu as pltpu

_SEQ, _HID, _FF, _E = 2048, 1024, 2048, 8
_CHUNK = _SEQ // _E
_LANES = 128
_FT = 512
_FN = _FF // _FT


def _moe_body(x_ref, lg_ref, wg_ref, wu_ref, wsg_ref, wsu_ref, wd_ref,
              wsd_ref, out_ref, sc_ref):
    c = pl.program_id(0)
    f = pl.program_id(1)
    rows = pl.ds(c * _CHUNK, _CHUNK)
    cols = pl.ds(f * _FT, _FT)

    l = lg_ref[rows, :]
    lane = jax.lax.broadcasted_iota(jnp.int32, l.shape, 1)
    neg_inf = jnp.float32(-jnp.inf)
    v1 = jnp.max(l, axis=1, keepdims=True)
    i1 = jnp.min(jnp.where(l == v1, lane, _LANES), axis=1, keepdims=True)
    l2 = jnp.where(lane == i1, neg_inf, l)
    v2 = jnp.max(l2, axis=1, keepdims=True)
    i2 = jnp.min(jnp.where(l2 == v2, lane, _LANES), axis=1, keepdims=True)
    s1 = jax.nn.sigmoid(v1)
    s2 = jax.nn.sigmoid(v2)

    sel = (lane == i1) | (lane == i2)
    sc_ref[...] = jnp.where(sel, jax.nn.sigmoid(l), 0.0).T

    x16 = x_ref[rows, :].astype(jnp.bfloat16)
    g = jnp.dot(x16, wg_ref[0].astype(jnp.bfloat16),
                preferred_element_type=jnp.float32)
    u = jnp.dot(x16, wu_ref[0].astype(jnp.bfloat16),
                preferred_element_type=jnp.float32)
    gs = jnp.dot(x16, wsg_ref[:, cols].astype(jnp.bfloat16),
                 preferred_element_type=jnp.float32)
    us = jnp.dot(x16, wsu_ref[:, cols].astype(jnp.bfloat16),
                 preferred_element_type=jnp.float32)

    a1 = s1 * g
    a2 = s2 * g
    h_routed = (s1 * u) * (a1 * jax.nn.sigmoid(a1)) \
             + (s2 * u) * (a2 * jax.nn.sigmoid(a2))
    h_shared = us * (gs * jax.nn.sigmoid(gs))

    partial = (
        jnp.dot(h_routed.astype(jnp.bfloat16), wd_ref[0].astype(jnp.bfloat16),
                preferred_element_type=jnp.float32)
        + jnp.dot(h_shared.astype(jnp.bfloat16),
                  wsd_ref[cols, :].astype(jnp.bfloat16),
                  preferred_element_type=jnp.float32))

    @pl.when(f == 0)
    def _():
        out_ref[...] = partial

    @pl.when(f > 0)
    def _():
        out_ref[...] += partial


def kernel(hidden_states, router_kernel, gate_up_proj, down_proj,
           shared_gate_kernel, shared_up_kernel, shared_down_kernel):
    batch, seq, hid = hidden_states.shape
    flat = hidden_states.reshape(seq, hid)
    router_logits = flat @ router_kernel

    out, scores_t = pl.pallas_call(
        _moe_body,
        grid=(_E, _FN),
        in_specs=[
            pl.BlockSpec((_SEQ, _HID), lambda c, f: (0, 0)),
            pl.BlockSpec((_SEQ, _E), lambda c, f: (0, 0)),
            pl.BlockSpec((1, _HID, _FT), lambda c, f: (c, 0, f)),
            pl.BlockSpec((1, _HID, _FT), lambda c, f: (c, 0, f + _FN)),
            pl.BlockSpec((_HID, _FF), lambda c, f: (0, 0)),
            pl.BlockSpec((_HID, _FF), lambda c, f: (0, 0)),
            pl.BlockSpec((1, _FT, _HID), lambda c, f: (c, f, 0)),
            pl.BlockSpec((_FF, _HID), lambda c, f: (0, 0)),
        ],
        out_specs=[
            pl.BlockSpec((_CHUNK, _HID), lambda c, f: (c, 0)),
            pl.BlockSpec((_E, _CHUNK), lambda c, f: (0, c)),
        ],
        out_shape=[
            jax.ShapeDtypeStruct((seq, hid), jnp.float32),
            jax.ShapeDtypeStruct((_E, seq), jnp.float32),
        ],
        compiler_params=pltpu.CompilerParams(
            dimension_semantics=("parallel", "arbitrary")),
    )(flat, router_logits, gate_up_proj, gate_up_proj,
      shared_gate_kernel, shared_up_kernel, down_proj, shared_down_kernel)

    return out.reshape(batch, seq, hid), scores_t
